# Initial kernel scaffold; baseline (speedup 1.0000x reference)
import jax
import jax.numpy as jnp
from jax.experimental import pallas as pl


def kernel(depths):
    raise NotImplementedError("write your pallas kernel here")


# speedup vs baseline: 26.2915x; 26.2915x over previous
import functools

import jax
import jax.numpy as jnp
from jax import lax
from jax.experimental import pallas as pl
from jax.experimental.pallas import tpu as pltpu
from jax.experimental.pallas import tpu_sc as plsc

_MIN_D, _MAX_D, _ND = 0.25, 10.0, 64
_INV = float((_ND - 1) / (_MAX_D - _MIN_D))
_W = 352
_NVREG_ROW = _W // 16
_CHUNK = 32 * _W
_ACC = 4 * 44 * _ND
_IMG_PIX = 128 * _W
_IMG_OUT = 924 * _ND
_P16_OFF = 704 * _ND
_P32_OFF = 880 * _ND


def _sc_body(depth_hbm, out_hbm, depth_v, acc_v, p16_v, p32_v):
    wid = lax.axis_index("s") * 2 + lax.axis_index("c")

    @pl.when(wid < 24)
    def _():
        b = wid >> 2
        g = wid & 3

        pltpu.sync_copy(
            depth_hbm.at[pl.ds(b * _IMG_PIX + g * _CHUNK, _CHUNK)], depth_v
        )

        zero16 = jnp.zeros((16,), jnp.float32)

        def zbody(z, carry):
            base = z * 256
            for u in range(16):
                acc_v[pl.ds(base + u * 16, 16)] = zero16
            return carry

        lax.fori_loop(0, _ACC // 256, zbody, None)

        lane = lax.iota(jnp.int32, 16)
        vco = (lane >> 3) << 6

        def rbody(r, carry):
            rowpix = r * _W
            rowbase = (r >> 3) * 2816
            for v in range(_NVREG_ROW):
                d = depth_v[pl.ds(rowpix + v * 16, 16)]
                d = jnp.minimum(jnp.maximum(d, _MIN_D), _MAX_D)
                t = (d - _MIN_D) * _INV
                lo = jnp.minimum(t.astype(jnp.int32), _ND - 2)
                frac = t - lo.astype(jnp.float32)
                whi = frac * (1.0 / 64.0)
                wlo = (1.0 / 64.0) - whi
                idx_lo = (vco + (rowbase + v * 128)) + lo
                plsc.addupdate_scatter(acc_v, [idx_lo], wlo)
                plsc.addupdate_scatter(acc_v, [idx_lo + 1], whi)
            return carry

        lax.fori_loop(0, 32, rbody, None)

        out_base = b * _IMG_OUT
        pltpu.sync_copy(acc_v, out_hbm.at[pl.ds(out_base + g * _ACC, _ACC)])

        def c2body(c2, carry):
            coff = c2 * 128
            for lr2 in range(2):
                o00 = (2 * lr2) * 44 * 64
                o10 = (2 * lr2 + 1) * 44 * 64
                for kc in range(4):
                    k = kc * 16
                    s = (
                        acc_v[pl.ds(o00 + coff + k, 16)]
                        + acc_v[pl.ds(o00 + coff + 64 + k, 16)]
                        + acc_v[pl.ds(o10 + coff + k, 16)]
                        + acc_v[pl.ds(o10 + coff + 64 + k, 16)]
                    )
                    p16_v[pl.ds(lr2 * 1408 + c2 * 64 + k, 16)] = s * 0.25
            return carry

        lax.fori_loop(0, 22, c2body, None)
        pltpu.sync_copy(
            p16_v, out_hbm.at[pl.ds(out_base + _P16_OFF + g * 2816, 2816)]
        )

        def c4body(c4, carry):
            coff = c4 * 128
            for kc in range(4):
                k = kc * 16
                s = (
                    p16_v[pl.ds(coff + k, 16)]
                    + p16_v[pl.ds(coff + 64 + k, 16)]
                    + p16_v[pl.ds(1408 + coff + k, 16)]
                    + p16_v[pl.ds(1408 + coff + 64 + k, 16)]
                )
                p32_v[pl.ds(c4 * 64 + k, 16)] = s * 0.25
            return carry

        lax.fori_loop(0, 11, c4body, None)
        pltpu.sync_copy(
            p32_v, out_hbm.at[pl.ds(out_base + _P32_OFF + g * 704, 704)]
        )


@functools.cache
def _sc_call():
    return functools.partial(
        pl.kernel,
        out_type=jax.ShapeDtypeStruct((6 * _IMG_OUT,), jnp.float32),
        mesh=plsc.VectorSubcoreMesh(core_axis_name="c", subcore_axis_name="s"),
        compiler_params=pltpu.CompilerParams(needs_layout_passes=False),
        scratch_types=[
            pltpu.VMEM((_CHUNK,), jnp.float32),
            pltpu.VMEM((_ACC,), jnp.float32),
            pltpu.VMEM((2816,), jnp.float32),
            pltpu.VMEM((704,), jnp.float32),
        ],
    )(_sc_body)


def kernel(depths):
    flat = depths.reshape(-1).astype(jnp.float32)
    out = _sc_call()(flat)
    return out.reshape(1, 6, 924, _ND)


# speedup vs baseline: 32.6457x; 1.2417x over previous
import functools

import jax
import jax.numpy as jnp
from jax import lax
from jax.experimental import pallas as pl
from jax.experimental.pallas import tpu as pltpu
from jax.experimental.pallas import tpu_sc as plsc

_MIN_D, _MAX_D, _ND = 0.25, 10.0, 64
_INV = float((_ND - 1) / (_MAX_D - _MIN_D))
_C2 = -_MIN_D * _INV
_W = 352
_NVREG_ROW = _W // 16
_CHUNK = 32 * _W
_ACC = 4 * 44 * _ND
_IMG_PIX = 128 * _W
_IMG_OUT = 924 * _ND
_P16_OFF = 704 * _ND
_P32_OFF = 880 * _ND


def _sc_body(depth_hbm, zeros_hbm, out_hbm, depth_v, acc_v, p16_v, p32_v,
             sem_a, sem_b):
    wid = lax.axis_index("s") * 2 + lax.axis_index("c")

    @pl.when(wid < 24)
    def _():
        b = wid >> 2
        g = wid & 3

        cp_d = pltpu.async_copy(
            depth_hbm.at[pl.ds(b * _IMG_PIX + g * _CHUNK, _CHUNK)],
            depth_v, sem_a,
        )
        cp_z = pltpu.async_copy(zeros_hbm, acc_v, sem_b)

        lane = lax.iota(jnp.int32, 16)
        vco = (lane >> 3) << 6

        cp_d.wait()
        cp_z.wait()

        def rbody(r, carry):
            rowpix = r * _W
            rowbase = (r >> 3) * 2816
            for g0, gn in ((0, 8), (8, 8), (16, 6)):
                ds = [depth_v[pl.ds(rowpix + (g0 + j) * 16, 16)]
                      for j in range(gn)]
                ts = [jnp.minimum(jnp.maximum(d * _INV + _C2, 0.0), 63.0)
                      for d in ds]
                los = [jnp.minimum(t, 62.5).astype(jnp.int32) for t in ts]
                fracs = [t - lo.astype(jnp.float32)
                         for t, lo in zip(ts, los)]
                whis = [f * (1.0 / 64.0) for f in fracs]
                wlos = [(1.0 / 64.0) - w for w in whis]
                idxs = [vco + lo for lo in los]
                for j in range(gn):
                    win = acc_v.at[pl.ds(rowbase + (g0 + j) * 128, 128)]
                    plsc.addupdate_scatter(win, [idxs[j]], wlos[j])
                    plsc.addupdate_scatter(win, [idxs[j] + 1], whis[j])
            return carry

        lax.fori_loop(0, 32, rbody, None)

        out_base = b * _IMG_OUT
        pltpu.sync_copy(acc_v, out_hbm.at[pl.ds(out_base + g * _ACC, _ACC)])

        def c2body(c2, carry):
            coff = c2 * 128
            for lr2 in range(2):
                o00 = (2 * lr2) * 44 * 64
                o10 = (2 * lr2 + 1) * 44 * 64
                for kc in range(4):
                    k = kc * 16
                    s = (
                        acc_v[pl.ds(o00 + coff + k, 16)]
                        + acc_v[pl.ds(o00 + coff + 64 + k, 16)]
                        + acc_v[pl.ds(o10 + coff + k, 16)]
                        + acc_v[pl.ds(o10 + coff + 64 + k, 16)]
                    )
                    p16_v[pl.ds(lr2 * 1408 + c2 * 64 + k, 16)] = s * 0.25
            return carry

        lax.fori_loop(0, 22, c2body, None)
        pltpu.sync_copy(
            p16_v, out_hbm.at[pl.ds(out_base + _P16_OFF + g * 2816, 2816)]
        )

        def c4body(c4, carry):
            coff = c4 * 128
            for kc in range(4):
                k = kc * 16
                s = (
                    p16_v[pl.ds(coff + k, 16)]
                    + p16_v[pl.ds(coff + 64 + k, 16)]
                    + p16_v[pl.ds(1408 + coff + k, 16)]
                    + p16_v[pl.ds(1408 + coff + 64 + k, 16)]
                )
                p32_v[pl.ds(c4 * 64 + k, 16)] = s * 0.25
            return carry

        lax.fori_loop(0, 11, c4body, None)
        pltpu.sync_copy(
            p32_v, out_hbm.at[pl.ds(out_base + _P32_OFF + g * 704, 704)]
        )


@functools.cache
def _sc_call():
    return functools.partial(
        pl.kernel,
        out_type=jax.ShapeDtypeStruct((6 * _IMG_OUT,), jnp.float32),
        mesh=plsc.VectorSubcoreMesh(core_axis_name="c", subcore_axis_name="s"),
        compiler_params=pltpu.CompilerParams(needs_layout_passes=False),
        scratch_types=[
            pltpu.VMEM((_CHUNK,), jnp.float32),
            pltpu.VMEM((_ACC,), jnp.float32),
            pltpu.VMEM((2816,), jnp.float32),
            pltpu.VMEM((704,), jnp.float32),
            pltpu.SemaphoreType.DMA,
            pltpu.SemaphoreType.DMA,
        ],
    )(_sc_body)


def kernel(depths):
    flat = depths.reshape(-1).astype(jnp.float32)
    zeros = jnp.zeros((_ACC,), jnp.float32)
    out = _sc_call()(flat, zeros)
    return out.reshape(1, 6, 924, _ND)


# speedup vs baseline: 34.6582x; 1.0616x over previous
import functools

import jax
import jax.numpy as jnp
from jax import lax
from jax.experimental import pallas as pl
from jax.experimental.pallas import tpu as pltpu
from jax.experimental.pallas import tpu_sc as plsc

_MIN_D, _MAX_D, _ND = 0.25, 10.0, 64
_INV = float((_ND - 1) / (_MAX_D - _MIN_D))
_C2 = -_MIN_D * _INV
_W = 352
_ACC = 4 * 44 * _ND
_IMG_OUT = 924 * _ND
_P16_OFF = 704 * _ND
_P32_OFF = 880 * _ND


def _sc_body(depth_hbm, out_hbm, depth_v, acc_v, p16_v, p32_v, sem_a):
    wid = lax.axis_index("s") * 2 + lax.axis_index("c")

    @pl.when(wid < 24)
    def _():
        b = wid >> 2
        g = wid & 3

        cp_d = pltpu.async_copy(
            depth_hbm.at[0, b, 0, pl.ds(g * 32, 32), :], depth_v, sem_a
        )

        zero16 = jnp.zeros((16,), jnp.float32)

        def zbody(z, carry):
            base = z * 256
            for u in range(16):
                acc_v[pl.ds(base + u * 16, 16)] = zero16
            return carry

        lax.fori_loop(0, _ACC // 256, zbody, None)

        lane = lax.iota(jnp.int32, 16)
        vco = (lane >> 3) << 6

        cp_d.wait()

        def rbody(r, carry):
            rowbase = (r >> 3) * 2816
            for g0, gn in ((0, 8), (8, 8), (16, 6)):
                ds = [depth_v[r, pl.ds((g0 + j) * 16, 16)]
                      for j in range(gn)]
                ts = [jnp.minimum(jnp.maximum(d * _INV + _C2, 0.0), 63.0)
                      for d in ds]
                los = [jnp.minimum(t, 62.5).astype(jnp.int32) for t in ts]
                fracs = [t - lo.astype(jnp.float32)
                         for t, lo in zip(ts, los)]
                whis = [f * (1.0 / 64.0) for f in fracs]
                wlos = [(1.0 / 64.0) - w for w in whis]
                idxs = [vco + lo for lo in los]
                for j in range(gn):
                    win = acc_v.at[pl.ds(rowbase + (g0 + j) * 128, 128)]
                    plsc.addupdate_scatter(win, [idxs[j]], wlos[j])
                    plsc.addupdate_scatter(win, [idxs[j] + 1], whis[j])
            return carry

        lax.fori_loop(0, 32, rbody, None)

        out_base = b * _IMG_OUT
        pltpu.sync_copy(acc_v, out_hbm.at[pl.ds(out_base + g * _ACC, _ACC)])

        def c2body(c2, carry):
            coff = c2 * 128
            for lr2 in range(2):
                o00 = (2 * lr2) * 44 * 64
                o10 = (2 * lr2 + 1) * 44 * 64
                for kc in range(4):
                    k = kc * 16
                    s = (
                        acc_v[pl.ds(o00 + coff + k, 16)]
                        + acc_v[pl.ds(o00 + coff + 64 + k, 16)]
                        + acc_v[pl.ds(o10 + coff + k, 16)]
                        + acc_v[pl.ds(o10 + coff + 64 + k, 16)]
                    )
                    p16_v[pl.ds(lr2 * 1408 + c2 * 64 + k, 16)] = s * 0.25
            return carry

        lax.fori_loop(0, 22, c2body, None)
        pltpu.sync_copy(
            p16_v, out_hbm.at[pl.ds(out_base + _P16_OFF + g * 2816, 2816)]
        )

        def c4body(c4, carry):
            coff = c4 * 128
            for kc in range(4):
                k = kc * 16
                s = (
                    p16_v[pl.ds(coff + k, 16)]
                    + p16_v[pl.ds(coff + 64 + k, 16)]
                    + p16_v[pl.ds(1408 + coff + k, 16)]
                    + p16_v[pl.ds(1408 + coff + 64 + k, 16)]
                )
                p32_v[pl.ds(c4 * 64 + k, 16)] = s * 0.25
            return carry

        lax.fori_loop(0, 11, c4body, None)
        pltpu.sync_copy(
            p32_v, out_hbm.at[pl.ds(out_base + _P32_OFF + g * 704, 704)]
        )


@functools.cache
def _sc_call():
    return functools.partial(
        pl.kernel,
        out_type=jax.ShapeDtypeStruct((6 * _IMG_OUT,), jnp.float32),
        mesh=plsc.VectorSubcoreMesh(core_axis_name="c", subcore_axis_name="s"),
        compiler_params=pltpu.CompilerParams(needs_layout_passes=False),
        scratch_types=[
            pltpu.VMEM((32, _W), jnp.float32),
            pltpu.VMEM((_ACC,), jnp.float32),
            pltpu.VMEM((2816,), jnp.float32),
            pltpu.VMEM((704,), jnp.float32),
            pltpu.SemaphoreType.DMA,
        ],
    )(_sc_body)


def kernel(depths):
    out = _sc_call()(depths)
    return out.reshape(1, 6, 924, _ND)


# speedup vs baseline: 34.6826x; 1.0007x over previous
import functools

import jax
import jax.numpy as jnp
from jax import lax
from jax.experimental import pallas as pl
from jax.experimental.pallas import tpu as pltpu
from jax.experimental.pallas import tpu_sc as plsc

_MIN_D, _MAX_D, _ND = 0.25, 10.0, 64
_INV = float((_ND - 1) / (_MAX_D - _MIN_D))
_C2 = -_MIN_D * _INV
_W = 352
_ACC = 4 * 44 * _ND
_IMG_OUT = 924 * _ND
_P16_OFF = 704 * _ND
_P32_OFF = 880 * _ND


def _sc_body(depth_hbm, out_hbm, depth_v, acc_v, p16_v, p32_v, sem_a):
    wid = lax.axis_index("s") * 2 + lax.axis_index("c")

    @pl.when(wid < 24)
    def _():
        b = wid >> 2
        g = wid & 3

        cp_d = pltpu.async_copy(
            depth_hbm.at[0, b, 0, pl.ds(g * 32, 32), :], depth_v, sem_a
        )

        zero16 = jnp.zeros((16,), jnp.float32)

        def zbody(z, carry):
            base = z * 256
            for u in range(16):
                acc_v[pl.ds(base + u * 16, 16)] = zero16
            return carry

        lax.fori_loop(0, _ACC // 256, zbody, None)

        lane = lax.iota(jnp.int32, 16)
        vco = (lane >> 3) << 6

        cp_d.wait()

        def rbody(r, carry):
            rowbase = (r >> 3) * 2816
            for g0, gn in ((0, 8), (8, 8), (16, 6)):
                ds = [depth_v[r, pl.ds((g0 + j) * 16, 16)]
                      for j in range(gn)]
                ts = [jnp.minimum(jnp.maximum(d * _INV + _C2, 0.0), 63.0)
                      for d in ds]
                los = [jnp.minimum(t, 62.5).astype(jnp.int32) for t in ts]
                fracs = [t - lo.astype(jnp.float32)
                         for t, lo in zip(ts, los)]
                whis = [f * (1.0 / 64.0) for f in fracs]
                wlos = [(1.0 / 64.0) - w for w in whis]
                idxs = [vco + lo for lo in los]
                for j in range(gn):
                    win = acc_v.at[pl.ds(rowbase + (g0 + j) * 128, 128)]
                    plsc.addupdate_scatter(win, [idxs[j]], wlos[j])
                    plsc.addupdate_scatter(win, [idxs[j] + 1], whis[j])
            return carry

        lax.fori_loop(0, 32, rbody, None)

        out_base = b * _IMG_OUT
        pltpu.sync_copy(acc_v, out_hbm.at[pl.ds(out_base + g * _ACC, _ACC)])

        def c2body(c2, carry):
            coff = c2 * 128
            for lr2 in range(2):
                o00 = (2 * lr2) * 44 * 64
                o10 = (2 * lr2 + 1) * 44 * 64
                for kc in range(4):
                    k = kc * 16
                    s = (
                        acc_v[pl.ds(o00 + coff + k, 16)]
                        + acc_v[pl.ds(o00 + coff + 64 + k, 16)]
                        + acc_v[pl.ds(o10 + coff + k, 16)]
                        + acc_v[pl.ds(o10 + coff + 64 + k, 16)]
                    )
                    p16_v[pl.ds(lr2 * 1408 + c2 * 64 + k, 16)] = s * 0.25
            return carry

        lax.fori_loop(0, 22, c2body, None)
        pltpu.sync_copy(
            p16_v, out_hbm.at[pl.ds(out_base + _P16_OFF + g * 2816, 2816)]
        )

        def c4body(c4, carry):
            coff = c4 * 128
            for kc in range(4):
                k = kc * 16
                s = (
                    p16_v[pl.ds(coff + k, 16)]
                    + p16_v[pl.ds(coff + 64 + k, 16)]
                    + p16_v[pl.ds(1408 + coff + k, 16)]
                    + p16_v[pl.ds(1408 + coff + 64 + k, 16)]
                )
                p32_v[pl.ds(c4 * 64 + k, 16)] = s * 0.25
            return carry

        lax.fori_loop(0, 11, c4body, None)
        pltpu.sync_copy(
            p32_v, out_hbm.at[pl.ds(out_base + _P32_OFF + g * 704, 704)]
        )


@functools.cache
def _sc_call():
    return functools.partial(
        pl.kernel,
        out_type=jax.ShapeDtypeStruct((6 * _IMG_OUT,), jnp.float32),
        mesh=plsc.VectorSubcoreMesh(core_axis_name="c", subcore_axis_name="s"),
        compiler_params=pltpu.CompilerParams(
            needs_layout_passes=False, use_tc_tiling_on_sc=True
        ),
        scratch_types=[
            pltpu.VMEM((32, _W), jnp.float32),
            pltpu.VMEM((_ACC,), jnp.float32),
            pltpu.VMEM((2816,), jnp.float32),
            pltpu.VMEM((704,), jnp.float32),
            pltpu.SemaphoreType.DMA,
        ],
    )(_sc_body)


def kernel(depths):
    out = _sc_call()(depths)
    return out.reshape(1, 6, 924, _ND)


# speedup vs baseline: 34.7059x; 1.0007x over previous
import functools

import jax
import jax.numpy as jnp
from jax import lax
from jax.experimental import pallas as pl
from jax.experimental.pallas import tpu as pltpu
from jax.experimental.pallas import tpu_sc as plsc

_MIN_D, _MAX_D, _ND = 0.25, 10.0, 64
_INV = float((_ND - 1) / (_MAX_D - _MIN_D))
_C2 = -_MIN_D * _INV
_W = 352
_ACC = 4 * 44 * _ND
_IMG_OUT = 924 * _ND
_P16_OFF = 704 * _ND
_P32_OFF = 880 * _ND


def _sc_body(depth_hbm, out_hbm, depth_v, acc_v, p16_v, p32_v, sem_a):
    wid = lax.axis_index("s") * 2 + lax.axis_index("c")

    @pl.when(wid < 24)
    def _():
        b = wid >> 2
        g = wid & 3

        cp_d = pltpu.async_copy(
            depth_hbm.at[0, b, 0, pl.ds(g * 32, 32), :], depth_v, sem_a
        )

        zero16 = jnp.zeros((16,), jnp.float32)

        def zbody(z, carry):
            base = z * 256
            for u in range(16):
                acc_v[pl.ds(base + u * 16, 16)] = zero16
            return carry

        lax.fori_loop(0, _ACC // 256, zbody, None)

        lane = lax.iota(jnp.int32, 16)
        vco = (lane >> 3) << 6

        cp_d.wait()

        def rbody(r, carry):
            rowbase = (r >> 3) * 2816
            for g0, gn in ((0, 8), (8, 8), (16, 6)):
                ds = [depth_v[r, pl.ds((g0 + j) * 16, 16)]
                      for j in range(gn)]
                ts = [jnp.minimum(jnp.maximum(d * _INV + _C2, 0.0), 63.0)
                      for d in ds]
                los = [jnp.minimum(t, 62.5).astype(jnp.int32) for t in ts]
                fracs = [t - lo.astype(jnp.float32)
                         for t, lo in zip(ts, los)]
                whis = [f * (1.0 / 64.0) for f in fracs]
                wlos = [(1.0 / 64.0) - w for w in whis]
                idxs = [vco + lo for lo in los]
                for j in range(gn):
                    win = acc_v.at[pl.ds(rowbase + (g0 + j) * 128, 128)]
                    plsc.addupdate_scatter(win, [idxs[j]], wlos[j])
                    plsc.addupdate_scatter(win, [idxs[j] + 1], whis[j])
            return carry

        lax.fori_loop(0, 32, rbody, None)

        out_base = b * _IMG_OUT
        pltpu.sync_copy(acc_v, out_hbm.at[pl.ds(out_base + g * _ACC, _ACC)])

        def c2body(c2, carry):
            coff = c2 * 128
            for lr2 in range(2):
                o00 = (2 * lr2) * 44 * 64
                o10 = (2 * lr2 + 1) * 44 * 64
                for kc in range(4):
                    k = kc * 16
                    s = (
                        acc_v[pl.ds(o00 + coff + k, 16)]
                        + acc_v[pl.ds(o00 + coff + 64 + k, 16)]
                        + acc_v[pl.ds(o10 + coff + k, 16)]
                        + acc_v[pl.ds(o10 + coff + 64 + k, 16)]
                    )
                    p16_v[pl.ds(lr2 * 1408 + c2 * 64 + k, 16)] = s * 0.25
            return carry

        lax.fori_loop(0, 22, c2body, None)
        pltpu.sync_copy(
            p16_v, out_hbm.at[pl.ds(out_base + _P16_OFF + g * 2816, 2816)]
        )

        def c4body(c4, carry):
            coff = c4 * 128
            for kc in range(4):
                k = kc * 16
                s = (
                    p16_v[pl.ds(coff + k, 16)]
                    + p16_v[pl.ds(coff + 64 + k, 16)]
                    + p16_v[pl.ds(1408 + coff + k, 16)]
                    + p16_v[pl.ds(1408 + coff + 64 + k, 16)]
                )
                p32_v[pl.ds(c4 * 64 + k, 16)] = s * 0.25
            return carry

        lax.fori_loop(0, 11, c4body, None)
        pltpu.sync_copy(
            p32_v, out_hbm.at[pl.ds(out_base + _P32_OFF + g * 704, 704)]
        )


@functools.cache
def _sc_call():
    return functools.partial(
        pl.kernel,
        out_type=jax.ShapeDtypeStruct((6 * _IMG_OUT,), jnp.float32),
        mesh=plsc.VectorSubcoreMesh(core_axis_name="c", subcore_axis_name="s"),
        compiler_params=pltpu.CompilerParams(
            needs_layout_passes=False, skip_device_barrier=True
        ),
        scratch_types=[
            pltpu.VMEM((32, _W), jnp.float32),
            pltpu.VMEM((_ACC,), jnp.float32),
            pltpu.VMEM((2816,), jnp.float32),
            pltpu.VMEM((704,), jnp.float32),
            pltpu.SemaphoreType.DMA,
        ],
    )(_sc_body)


def kernel(depths):
    out = _sc_call()(depths)
    return out.reshape(1, 6, 924, _ND)


# speedup vs baseline: 34.8774x; 1.0049x over previous
import functools

import jax
import jax.numpy as jnp
from jax import lax
from jax.experimental import pallas as pl
from jax.experimental.pallas import tpu as pltpu
from jax.experimental.pallas import tpu_sc as plsc

_MIN_D, _MAX_D, _ND = 0.25, 10.0, 64
_INV = float((_ND - 1) / (_MAX_D - _MIN_D))
_C2 = -_MIN_D * _INV
_W = 352
_ACC = 4 * 44 * _ND
_IMG_OUT = 924 * _ND
_P16_OFF = 704 * _ND
_P32_OFF = 880 * _ND


def _sc_body(depth_hbm, out_hbm, depth_v, acc_v, p16_v, p32_v, sem_a):
    wid = lax.axis_index("s") * 2 + lax.axis_index("c")

    @pl.when(wid < 24)
    def _():
        b = wid >> 2
        g = wid & 3

        cp_d = pltpu.async_copy(
            depth_hbm.at[0, b, 0, pl.ds(g * 32, 32), :], depth_v, sem_a
        )

        zero16 = jnp.zeros((16,), jnp.float32)

        def zbody(z, carry):
            base = z * 256
            for u in range(16):
                acc_v[pl.ds(base + u * 16, 16)] = zero16
            return carry

        lax.fori_loop(0, _ACC // 256, zbody, None)

        lane = lax.iota(jnp.int32, 16)
        vco = (lane >> 3) << 6

        cp_d.wait()

        def rbody(r, carry):
            rowbase = (r >> 3) * 2816
            for g0, gn in ((0, 11), (11, 11)):
                ds = [depth_v[r, pl.ds((g0 + j) * 16, 16)]
                      for j in range(gn)]
                ts = [jnp.minimum(jnp.maximum(d * _INV + _C2, 0.0), 63.0)
                      for d in ds]
                los = [jnp.minimum(t, 62.5).astype(jnp.int32) for t in ts]
                fracs = [t - lo.astype(jnp.float32)
                         for t, lo in zip(ts, los)]
                whis = [f * (1.0 / 64.0) for f in fracs]
                wlos = [(1.0 / 64.0) - w for w in whis]
                idxs = [vco + lo for lo in los]
                for j in range(gn):
                    win = acc_v.at[pl.ds(rowbase + (g0 + j) * 128, 128)]
                    plsc.addupdate_scatter(win, [idxs[j]], wlos[j])
                    plsc.addupdate_scatter(win, [idxs[j] + 1], whis[j])
            return carry

        lax.fori_loop(0, 32, rbody, None)

        out_base = b * _IMG_OUT
        pltpu.sync_copy(acc_v, out_hbm.at[pl.ds(out_base + g * _ACC, _ACC)])

        def c2body(c2, carry):
            coff = c2 * 128
            for lr2 in range(2):
                o00 = (2 * lr2) * 44 * 64
                o10 = (2 * lr2 + 1) * 44 * 64
                for kc in range(4):
                    k = kc * 16
                    s = (
                        acc_v[pl.ds(o00 + coff + k, 16)]
                        + acc_v[pl.ds(o00 + coff + 64 + k, 16)]
                        + acc_v[pl.ds(o10 + coff + k, 16)]
                        + acc_v[pl.ds(o10 + coff + 64 + k, 16)]
                    )
                    p16_v[pl.ds(lr2 * 1408 + c2 * 64 + k, 16)] = s * 0.25
            return carry

        lax.fori_loop(0, 22, c2body, None)
        pltpu.sync_copy(
            p16_v, out_hbm.at[pl.ds(out_base + _P16_OFF + g * 2816, 2816)]
        )

        def c4body(c4, carry):
            coff = c4 * 128
            for kc in range(4):
                k = kc * 16
                s = (
                    p16_v[pl.ds(coff + k, 16)]
                    + p16_v[pl.ds(coff + 64 + k, 16)]
                    + p16_v[pl.ds(1408 + coff + k, 16)]
                    + p16_v[pl.ds(1408 + coff + 64 + k, 16)]
                )
                p32_v[pl.ds(c4 * 64 + k, 16)] = s * 0.25
            return carry

        lax.fori_loop(0, 11, c4body, None)
        pltpu.sync_copy(
            p32_v, out_hbm.at[pl.ds(out_base + _P32_OFF + g * 704, 704)]
        )


@functools.cache
def _sc_call():
    return functools.partial(
        pl.kernel,
        out_type=jax.ShapeDtypeStruct((6 * _IMG_OUT,), jnp.float32),
        mesh=plsc.VectorSubcoreMesh(core_axis_name="c", subcore_axis_name="s"),
        compiler_params=pltpu.CompilerParams(needs_layout_passes=False),
        scratch_types=[
            pltpu.VMEM((32, _W), jnp.float32),
            pltpu.VMEM((_ACC,), jnp.float32),
            pltpu.VMEM((2816,), jnp.float32),
            pltpu.VMEM((704,), jnp.float32),
            pltpu.SemaphoreType.DMA,
        ],
    )(_sc_body)


def kernel(depths):
    out = _sc_call()(depths)
    return out.reshape(1, 6, 924, _ND)


# speedup vs baseline: 36.5283x; 1.0473x over previous
import functools

import jax
import jax.numpy as jnp
from jax import lax
from jax.experimental import pallas as pl
from jax.experimental.pallas import tpu as pltpu
from jax.experimental.pallas import tpu_sc as plsc

_MIN_D, _MAX_D, _ND = 0.25, 10.0, 64
_INV = float((_ND - 1) / (_MAX_D - _MIN_D))
_C2 = -_MIN_D * _INV
_W = 352
_ACC = 4 * 44 * _ND
_IMG_OUT = 924 * _ND
_P16_OFF = 704 * _ND
_P32_OFF = 880 * _ND


def _sc_body(depth_hbm, out_hbm, depth_v, acc_v, p16_v, p32_v, sem_a):
    wid = lax.axis_index("s") * 2 + lax.axis_index("c")

    @pl.when(wid < 24)
    def _():
        b = wid >> 2
        g = wid & 3

        cp_d = pltpu.async_copy(
            depth_hbm.at[0, b, 0, pl.ds(g * 32, 32), :], depth_v, sem_a
        )

        zero16 = jnp.zeros((16,), jnp.float32)

        def zbody(z, carry):
            base = z * 256
            for u in range(16):
                acc_v[pl.ds(base + u * 16, 16)] = zero16
            return carry

        lax.fori_loop(0, _ACC // 256, zbody, None)

        lane = lax.iota(jnp.int32, 16)
        vco = (lane >> 3) << 6

        cp_d.wait()

        def rbody(r, carry):
            rowbase = (r >> 3) * 2816
            for g0, gn in ((0, 11), (11, 11)):
                ds = [depth_v[r, pl.ds((g0 + j) * 16, 16)]
                      for j in range(gn)]
                ts = [jnp.minimum(jnp.maximum(d * _INV + _C2, 0.0), 63.0)
                      for d in ds]
                los = [jnp.minimum(t, 62.5).astype(jnp.int32) for t in ts]
                fracs = [t - lo.astype(jnp.float32)
                         for t, lo in zip(ts, los)]
                whis = [f * (1.0 / 64.0) for f in fracs]
                wlos = [(1.0 / 64.0) - w for w in whis]
                idxs = [vco + lo for lo in los]
                for j in range(gn):
                    win = acc_v.at[pl.ds(rowbase + (g0 + j) * 128, 128)]
                    plsc.addupdate_scatter(win, [idxs[j]], wlos[j])
                    plsc.addupdate_scatter(win, [idxs[j] + 1], whis[j])
            return carry

        lax.fori_loop(0, 32, rbody, None)

        out_base = b * _IMG_OUT
        pltpu.sync_copy(acc_v, out_hbm.at[pl.ds(out_base + g * _ACC, _ACC)])

        def c2body(c2, carry):
            coff = c2 * 128
            chains = []
            for lr2 in range(2):
                o00 = (2 * lr2) * 44 * 64
                o10 = (2 * lr2 + 1) * 44 * 64
                for kc in range(4):
                    k = kc * 16
                    chains.append((lr2, k, o00 + coff + k, o10 + coff + k))
            lds = [(acc_v[pl.ds(oa, 16)], acc_v[pl.ds(oa + 64, 16)],
                    acc_v[pl.ds(ob, 16)], acc_v[pl.ds(ob + 64, 16)])
                   for (_, _, oa, ob) in chains]
            s1 = [a + bb for (a, bb, _, _) in lds]
            s2 = [c + dd for (_, _, c, dd) in lds]
            s3 = [a + bb for a, bb in zip(s1, s2)]
            res = [s * 0.25 for s in s3]
            for (lr2, k, _, _), v in zip(chains, res):
                p16_v[pl.ds(lr2 * 1408 + c2 * 64 + k, 16)] = v
            return carry

        lax.fori_loop(0, 22, c2body, None)
        pltpu.sync_copy(
            p16_v, out_hbm.at[pl.ds(out_base + _P16_OFF + g * 2816, 2816)]
        )

        def c4body(c4, carry):
            coff = c4 * 128
            ks = [kc * 16 for kc in range(4)]
            lds = [(p16_v[pl.ds(coff + k, 16)],
                    p16_v[pl.ds(coff + 64 + k, 16)],
                    p16_v[pl.ds(1408 + coff + k, 16)],
                    p16_v[pl.ds(1408 + coff + 64 + k, 16)]) for k in ks]
            s1 = [a + bb for (a, bb, _, _) in lds]
            s2 = [c + dd for (_, _, c, dd) in lds]
            s3 = [a + bb for a, bb in zip(s1, s2)]
            res = [s * 0.25 for s in s3]
            for k, v in zip(ks, res):
                p32_v[pl.ds(c4 * 64 + k, 16)] = v
            return carry

        lax.fori_loop(0, 11, c4body, None)
        pltpu.sync_copy(
            p32_v, out_hbm.at[pl.ds(out_base + _P32_OFF + g * 704, 704)]
        )


@functools.cache
def _sc_call():
    return functools.partial(
        pl.kernel,
        out_type=jax.ShapeDtypeStruct((6 * _IMG_OUT,), jnp.float32),
        mesh=plsc.VectorSubcoreMesh(core_axis_name="c", subcore_axis_name="s"),
        compiler_params=pltpu.CompilerParams(needs_layout_passes=False),
        scratch_types=[
            pltpu.VMEM((32, _W), jnp.float32),
            pltpu.VMEM((_ACC,), jnp.float32),
            pltpu.VMEM((2816,), jnp.float32),
            pltpu.VMEM((704,), jnp.float32),
            pltpu.SemaphoreType.DMA,
        ],
    )(_sc_body)


def kernel(depths):
    out = _sc_call()(depths)
    return out.reshape(1, 6, 924, _ND)
